# baseline (speedup 1.0000x reference)
from functools import partial

import jax
import jax.numpy as jnp
from jax.experimental import pallas as pl
from jax.experimental.pallas import tpu as pltpu
from jax.experimental.pallas import tpu_sc as plsc

_B, _S, _D, _E, _V = 2, 2048, 768, 8, 100000
_N = _B * _S
_KB_IMG = 3072
_NB_IMG = 49
_KB_AUD = 3200
_NB_AUD = 5
_SB = 1024
_EPAD = 128

_HI = jax.lax.Precision.HIGHEST


def _sc_gather(tok_emb, idx):
    n = idx.shape[1]
    d = tok_emb.shape[1]
    n_units = 2 * 16
    per_unit = n // n_units
    chunk = 64
    mesh = plsc.VectorSubcoreMesh(core_axis_name="c", subcore_axis_name="s")

    @partial(
        pl.kernel,
        out_type=jax.ShapeDtypeStruct((n, d), tok_emb.dtype),
        mesh=mesh,
        scratch_types=[
            pltpu.VMEM((1, per_unit), jnp.int32),
            pltpu.VMEM((chunk, d), jnp.float32),
        ],
    )
    def gather_kernel(emb_hbm, idx_hbm, out_hbm, idx_vmem, buf):
        c = jax.lax.axis_index("c")
        s = jax.lax.axis_index("s")
        base = (c * 16 + s) * per_unit
        pltpu.sync_copy(idx_hbm.at[:, pl.ds(base, per_unit)], idx_vmem)

        @pl.loop(0, per_unit // chunk)
        def _(j):
            pltpu.sync_copy(emb_hbm.at[idx_vmem.at[0, pl.ds(j * chunk, chunk)]],
                            buf)
            pltpu.sync_copy(buf, out_hbm.at[pl.ds(base + j * chunk, chunk), :])

    return gather_kernel(tok_emb, idx)


def _proj_body(img_ref, wimg_ref, aud_ref, waud_ref, bias_ref, out_ref):
    i = pl.program_id(0)

    @pl.when(i == 0)
    def _init():
        out_ref[...] = jnp.broadcast_to(bias_ref[...], (_B, _D))

    @pl.when(i < _NB_IMG)
    def _img():
        out_ref[...] += jax.lax.dot_general(
            img_ref[...], wimg_ref[...], (((1,), (0,)), ((), ())),
            precision=_HI, preferred_element_type=jnp.float32)

    @pl.when(i >= _NB_IMG)
    def _aud():
        out_ref[...] += jax.lax.dot_general(
            aud_ref[...], waud_ref[...], (((1,), (0,)), ((), ())),
            precision=_HI, preferred_element_type=jnp.float32)


def _proj(img, W_img, aud, W_aud, bias_row):
    grid = (_NB_IMG + _NB_AUD,)
    return pl.pallas_call(
        _proj_body,
        grid=grid,
        in_specs=[
            pl.BlockSpec((_B, _KB_IMG), lambda i: (0, jnp.minimum(i, _NB_IMG - 1))),
            pl.BlockSpec((_KB_IMG, _D), lambda i: (jnp.minimum(i, _NB_IMG - 1), 0)),
            pl.BlockSpec((_B, _KB_AUD), lambda i: (0, jnp.maximum(i - _NB_IMG, 0))),
            pl.BlockSpec((_KB_AUD, _D), lambda i: (jnp.maximum(i - _NB_IMG, 0), 0)),
            pl.BlockSpec((1, _D), lambda i: (0, 0)),
        ],
        out_specs=pl.BlockSpec((_B, _D), lambda i: (0, 0)),
        out_shape=jax.ShapeDtypeStruct((_B, _D), jnp.float32),
    )(img, W_img, aud, W_aud, bias_row)


def _moe_body(t_ref, m_ref, c_ref, wr_ref, wexp_ref, bexp_ref, out_ref):
    i = pl.program_id(0)
    m = m_ref[:, 0:1]
    cval = c_ref[...]
    b_idx = (i * _SB) // _S
    crow = jnp.where(b_idx == 0, cval[0:1, :], cval[1:2, :])
    x = t_ref[...] * m + crow

    logits = jax.lax.dot_general(
        x, wr_ref[...], (((1,), (0,)), ((), ())),
        precision=_HI, preferred_element_type=jnp.float32)
    col = jax.lax.broadcasted_iota(jnp.int32, (_SB, _EPAD), 1)
    valid = col < _E
    neg = jnp.where(valid, logits, -jnp.inf)
    mx = jnp.max(neg, axis=1, keepdims=True)
    ex = jnp.where(valid, jnp.exp(neg - mx), 0.0)
    gates = ex / jnp.sum(ex, axis=1, keepdims=True)

    acc = jax.lax.dot_general(
        gates, bexp_ref[...], (((1,), (0,)), ((), ())),
        precision=_HI, preferred_element_type=jnp.float32)
    for e in range(_E):
        y = jax.lax.dot_general(
            x, wexp_ref[e], (((1,), (0,)), ((), ())),
            precision=_HI, preferred_element_type=jnp.float32)
        acc = acc + gates[:, e:e + 1] * y
    out_ref[...] = acc * m


def _moe(t, maskb, c, wr_pad, wexp, bexp_pad):
    grid = (_N // _SB,)
    return pl.pallas_call(
        _moe_body,
        grid=grid,
        in_specs=[
            pl.BlockSpec((_SB, _D), lambda i: (i, 0)),
            pl.BlockSpec((_SB, _EPAD), lambda i: (i, 0)),
            pl.BlockSpec((_B, _D), lambda i: (0, 0)),
            pl.BlockSpec((_D, _EPAD), lambda i: (0, 0)),
            pl.BlockSpec((_E, _D, _D), lambda i: (0, 0, 0)),
            pl.BlockSpec((_EPAD, _D), lambda i: (0, 0)),
        ],
        out_specs=pl.BlockSpec((_SB, _D), lambda i: (i, 0)),
        out_shape=jax.ShapeDtypeStruct((_N, _D), jnp.float32),
    )(t, maskb, c, wr_pad, wexp, bexp_pad)


def kernel(text_input, attention_mask, image_input, audio_input, tok_emb,
           W_img, b_img, W_aud, b_aud, W_router, W_experts, b_experts):
    idx = text_input.reshape(1, _N).astype(jnp.int32)
    t = _sc_gather(tok_emb, idx)

    img = image_input.reshape(_B, -1)
    bias_row = (b_img + b_aud).reshape(1, _D)
    c = _proj(img, W_img, audio_input, W_aud, bias_row)

    maskb = jnp.broadcast_to(
        attention_mask.astype(jnp.float32).reshape(_N, 1), (_N, _EPAD))
    wr_pad = jnp.zeros((_D, _EPAD), jnp.float32).at[:, :_E].set(W_router)
    bexp_pad = jnp.zeros((_EPAD, _D), jnp.float32).at[:_E, :].set(b_experts)

    out = _moe(t, maskb, c, wr_pad, W_experts, bexp_pad)
    return out.reshape(_B, _S, _D)


# speedup vs baseline: 1.7008x; 1.7008x over previous
from functools import partial

import jax
import jax.numpy as jnp
from jax.experimental import pallas as pl
from jax.experimental.pallas import tpu as pltpu
from jax.experimental.pallas import tpu_sc as plsc

_B, _S, _D, _E, _V = 2, 2048, 768, 8, 100000
_N = _B * _S
_KB_IMG = 3072
_NB_IMG = 49
_KB_AUD = 3200
_NB_AUD = 5
_SB = 1024
_EPAD = 128

_HI = jax.lax.Precision.HIGHEST


def _sc_gather(tok_emb, idx):
    n = idx.shape[1]
    d = tok_emb.shape[1]
    n_units = 2 * 16
    per_unit = n // n_units
    chunk = 64
    mesh = plsc.VectorSubcoreMesh(core_axis_name="c", subcore_axis_name="s")

    @partial(
        pl.kernel,
        out_type=jax.ShapeDtypeStruct((n, d), tok_emb.dtype),
        mesh=mesh,
        scratch_types=[
            pltpu.VMEM((1, per_unit), jnp.int32),
            pltpu.VMEM((chunk, d), jnp.float32),
        ],
    )
    def gather_kernel(emb_hbm, idx_hbm, out_hbm, idx_vmem, buf):
        c = jax.lax.axis_index("c")
        s = jax.lax.axis_index("s")
        base = (c * 16 + s) * per_unit
        pltpu.sync_copy(idx_hbm.at[:, pl.ds(base, per_unit)], idx_vmem)

        @pl.loop(0, per_unit // chunk)
        def _(j):
            pltpu.sync_copy(emb_hbm.at[idx_vmem.at[0, pl.ds(j * chunk, chunk)]],
                            buf)
            pltpu.sync_copy(buf, out_hbm.at[pl.ds(base + j * chunk, chunk), :])

    return gather_kernel(tok_emb, idx)


def _proj_body(img_ref, wimg_ref, aud_ref, waud_ref, bias_ref, out_ref):
    i = pl.program_id(0)

    @pl.when(i == 0)
    def _init():
        out_ref[...] = jnp.broadcast_to(bias_ref[...], (_B, _D))

    @pl.when(i < _NB_IMG)
    def _img():
        out_ref[...] += jax.lax.dot_general(
            img_ref[...], wimg_ref[...], (((1,), (0,)), ((), ())),
            precision=_HI, preferred_element_type=jnp.float32)

    @pl.when(i >= _NB_IMG)
    def _aud():
        out_ref[...] += jax.lax.dot_general(
            aud_ref[...], waud_ref[...], (((1,), (0,)), ((), ())),
            precision=_HI, preferred_element_type=jnp.float32)


def _proj(img, W_img, aud, W_aud, bias_row):
    grid = (_NB_IMG + _NB_AUD,)
    return pl.pallas_call(
        _proj_body,
        grid=grid,
        in_specs=[
            pl.BlockSpec((_B, _KB_IMG), lambda i: (0, jnp.minimum(i, _NB_IMG - 1))),
            pl.BlockSpec((_KB_IMG, _D), lambda i: (jnp.minimum(i, _NB_IMG - 1), 0)),
            pl.BlockSpec((_B, _KB_AUD), lambda i: (0, jnp.maximum(i - _NB_IMG, 0))),
            pl.BlockSpec((_KB_AUD, _D), lambda i: (jnp.maximum(i - _NB_IMG, 0), 0)),
            pl.BlockSpec((1, _D), lambda i: (0, 0)),
        ],
        out_specs=pl.BlockSpec((_B, _D), lambda i: (0, 0)),
        out_shape=jax.ShapeDtypeStruct((_B, _D), jnp.float32),
    )(img, W_img, aud, W_aud, bias_row)


def _moe_body(t_ref, m_ref, c_ref, wr_ref, wexp_ref, bexp_ref, out_ref):
    i = pl.program_id(0)
    m = m_ref[:, 0:1]
    cval = c_ref[...]
    b_idx = (i * _SB) // _S
    crow = jnp.where(b_idx == 0, cval[0:1, :], cval[1:2, :])
    x = t_ref[...] * m + crow

    logits = jax.lax.dot_general(
        x, wr_ref[...], (((1,), (0,)), ((), ())),
        precision=_HI, preferred_element_type=jnp.float32)
    col = jax.lax.broadcasted_iota(jnp.int32, (_SB, _EPAD), 1)
    valid = col < _E
    neg = jnp.where(valid, logits, -jnp.inf)
    mx = jnp.max(neg, axis=1, keepdims=True)
    ex = jnp.where(valid, jnp.exp(neg - mx), 0.0)
    gates = ex / jnp.sum(ex, axis=1, keepdims=True)

    acc = jax.lax.dot_general(
        gates, bexp_ref[...], (((1,), (0,)), ((), ())),
        precision=_HI, preferred_element_type=jnp.float32)
    x_bf = x.astype(jnp.bfloat16)
    for e in range(_E):
        y = jax.lax.dot_general(
            x_bf, wexp_ref[e], (((1,), (0,)), ((), ())),
            preferred_element_type=jnp.float32)
        acc = acc + gates[:, e:e + 1] * y
    out_ref[...] = acc * m


def _moe(t, maskb, c, wr_pad, wexp, bexp_pad):
    grid = (_N // _SB,)
    return pl.pallas_call(
        _moe_body,
        grid=grid,
        in_specs=[
            pl.BlockSpec((_SB, _D), lambda i: (i, 0)),
            pl.BlockSpec((_SB, _EPAD), lambda i: (i, 0)),
            pl.BlockSpec((_B, _D), lambda i: (0, 0)),
            pl.BlockSpec((_D, _EPAD), lambda i: (0, 0)),
            pl.BlockSpec((_E, _D, _D), lambda i: (0, 0, 0)),
            pl.BlockSpec((_EPAD, _D), lambda i: (0, 0)),
        ],
        out_specs=pl.BlockSpec((_SB, _D), lambda i: (i, 0)),
        out_shape=jax.ShapeDtypeStruct((_N, _D), jnp.float32),
    )(t, maskb, c, wr_pad, wexp, bexp_pad)


def kernel(text_input, attention_mask, image_input, audio_input, tok_emb,
           W_img, b_img, W_aud, b_aud, W_router, W_experts, b_experts):
    idx = text_input.reshape(1, _N).astype(jnp.int32)
    t = _sc_gather(tok_emb, idx)

    img = image_input.reshape(_B, -1)
    bias_row = (b_img + b_aud).reshape(1, _D)
    c = _proj(img, W_img, audio_input, W_aud, bias_row)

    maskb = jnp.broadcast_to(
        attention_mask.astype(jnp.float32).reshape(_N, 1), (_N, _EPAD))
    wr_pad = jnp.zeros((_D, _EPAD), jnp.float32).at[:, :_E].set(W_router)
    bexp_pad = jnp.zeros((_EPAD, _D), jnp.float32).at[:_E, :].set(b_experts)

    out = _moe(t, maskb, c, wr_pad, W_experts.astype(jnp.bfloat16), bexp_pad)
    return out.reshape(_B, _S, _D)


# speedup vs baseline: 1.7637x; 1.0370x over previous
from functools import partial

import jax
import jax.numpy as jnp
from jax.experimental import pallas as pl
from jax.experimental.pallas import tpu as pltpu
from jax.experimental.pallas import tpu_sc as plsc

_B, _S, _D, _E, _V = 2, 2048, 768, 8, 100000
_N = _B * _S
_KB_IMG = 3072
_NB_IMG = 49
_KB_AUD = 3200
_NB_AUD = 5
_SB = 1024
_EPAD = 128

_HI = jax.lax.Precision.HIGHEST


def _sc_gather(tok_emb, idx):
    n = idx.shape[1]
    d = tok_emb.shape[1]
    n_units = 2 * 16
    per_unit = n // n_units
    chunk = 64
    mesh = plsc.VectorSubcoreMesh(core_axis_name="c", subcore_axis_name="s")

    @partial(
        pl.kernel,
        out_type=jax.ShapeDtypeStruct((n, d), tok_emb.dtype),
        mesh=mesh,
        scratch_types=[
            pltpu.VMEM((1, per_unit), jnp.int32),
            pltpu.VMEM((chunk, d), jnp.float32),
        ],
    )
    def gather_kernel(emb_hbm, idx_hbm, out_hbm, idx_vmem, buf):
        c = jax.lax.axis_index("c")
        s = jax.lax.axis_index("s")
        base = (c * 16 + s) * per_unit
        pltpu.sync_copy(idx_hbm.at[:, pl.ds(base, per_unit)], idx_vmem)

        @pl.loop(0, per_unit // chunk)
        def _(j):
            pltpu.sync_copy(emb_hbm.at[idx_vmem.at[0, pl.ds(j * chunk, chunk)]],
                            buf)
            pltpu.sync_copy(buf, out_hbm.at[pl.ds(base + j * chunk, chunk), :])

    return gather_kernel(tok_emb, idx)


def _proj_body(imgT_ref, wimg_ref, audT_ref, waud_ref, bias_ref, out_ref):
    i = pl.program_id(0)

    @pl.when(i == 0)
    def _init():
        out_ref[...] = jnp.broadcast_to(bias_ref[...], (_B, _D))

    @pl.when(i < _NB_IMG)
    def _img():
        w = wimg_ref[...]
        it = imgT_ref[...]
        for b in range(_B):
            out_ref[b:b + 1, :] += jnp.sum(
                w * it[:, b:b + 1], axis=0, keepdims=True)

    @pl.when(i >= _NB_IMG)
    def _aud():
        w = waud_ref[...]
        at = audT_ref[...]
        for b in range(_B):
            out_ref[b:b + 1, :] += jnp.sum(
                w * at[:, b:b + 1], axis=0, keepdims=True)


def _proj(imgT, W_img, audT, W_aud, bias_row):
    grid = (_NB_IMG + _NB_AUD,)
    return pl.pallas_call(
        _proj_body,
        grid=grid,
        in_specs=[
            pl.BlockSpec((_KB_IMG, _B), lambda i: (jnp.minimum(i, _NB_IMG - 1), 0)),
            pl.BlockSpec((_KB_IMG, _D), lambda i: (jnp.minimum(i, _NB_IMG - 1), 0)),
            pl.BlockSpec((_KB_AUD, _B), lambda i: (jnp.maximum(i - _NB_IMG, 0), 0)),
            pl.BlockSpec((_KB_AUD, _D), lambda i: (jnp.maximum(i - _NB_IMG, 0), 0)),
            pl.BlockSpec((1, _D), lambda i: (0, 0)),
        ],
        out_specs=pl.BlockSpec((_B, _D), lambda i: (0, 0)),
        out_shape=jax.ShapeDtypeStruct((_B, _D), jnp.float32),
    )(imgT, W_img, audT, W_aud, bias_row)


def _moe_body(t_ref, m_ref, c_ref, wr_ref, wexp_ref, bexp_ref, out_ref):
    i = pl.program_id(0)
    m = m_ref[:, 0:1]
    cval = c_ref[...]
    b_idx = (i * _SB) // _S
    crow = jnp.where(b_idx == 0, cval[0:1, :], cval[1:2, :])
    x = t_ref[...] * m + crow

    logits = jax.lax.dot_general(
        x, wr_ref[...], (((1,), (0,)), ((), ())),
        precision=_HI, preferred_element_type=jnp.float32)
    col = jax.lax.broadcasted_iota(jnp.int32, (_SB, _EPAD), 1)
    valid = col < _E
    neg = jnp.where(valid, logits, -jnp.inf)
    mx = jnp.max(neg, axis=1, keepdims=True)
    ex = jnp.where(valid, jnp.exp(neg - mx), 0.0)
    gates = ex / jnp.sum(ex, axis=1, keepdims=True)

    acc = jax.lax.dot_general(
        gates, bexp_ref[...], (((1,), (0,)), ((), ())),
        precision=_HI, preferred_element_type=jnp.float32)
    x_bf = x.astype(jnp.bfloat16)
    for e in range(_E):
        y = jax.lax.dot_general(
            x_bf, wexp_ref[e], (((1,), (0,)), ((), ())),
            preferred_element_type=jnp.float32)
        acc = acc + gates[:, e:e + 1] * y
    out_ref[...] = acc * m


def _moe(t, maskb, c, wr_pad, wexp, bexp_pad):
    grid = (_N // _SB,)
    return pl.pallas_call(
        _moe_body,
        grid=grid,
        in_specs=[
            pl.BlockSpec((_SB, _D), lambda i: (i, 0)),
            pl.BlockSpec((_SB, _EPAD), lambda i: (i, 0)),
            pl.BlockSpec((_B, _D), lambda i: (0, 0)),
            pl.BlockSpec((_D, _EPAD), lambda i: (0, 0)),
            pl.BlockSpec((_E, _D, _D), lambda i: (0, 0, 0)),
            pl.BlockSpec((_EPAD, _D), lambda i: (0, 0)),
        ],
        out_specs=pl.BlockSpec((_SB, _D), lambda i: (i, 0)),
        out_shape=jax.ShapeDtypeStruct((_N, _D), jnp.float32),
    )(t, maskb, c, wr_pad, wexp, bexp_pad)


def kernel(text_input, attention_mask, image_input, audio_input, tok_emb,
           W_img, b_img, W_aud, b_aud, W_router, W_experts, b_experts):
    idx = text_input.reshape(1, _N).astype(jnp.int32)
    t = _sc_gather(tok_emb, idx)

    imgT = image_input.reshape(_B, -1).T
    audT = audio_input.T
    bias_row = (b_img + b_aud).reshape(1, _D)
    c = _proj(imgT, W_img, audT, W_aud, bias_row)

    maskb = jnp.broadcast_to(
        attention_mask.astype(jnp.float32).reshape(_N, 1), (_N, _EPAD))
    wr_pad = jnp.zeros((_D, _EPAD), jnp.float32).at[:, :_E].set(W_router)
    bexp_pad = jnp.zeros((_EPAD, _D), jnp.float32).at[:_E, :].set(b_experts)

    out = _moe(t, maskb, c, wr_pad, W_experts.astype(jnp.bfloat16), bexp_pad)
    return out.reshape(_B, _S, _D)


# speedup vs baseline: 2.1143x; 1.1988x over previous
from functools import partial

import jax
import jax.numpy as jnp
from jax.experimental import pallas as pl
from jax.experimental.pallas import tpu as pltpu
from jax.experimental.pallas import tpu_sc as plsc

_B, _S, _D, _E, _V = 2, 2048, 768, 8, 100000
_N = _B * _S
_KB_IMG = 3072
_NB_IMG = 49
_KB_AUD = 3200
_NB_AUD = 5
_SB = 1024
_EPAD = 128

_HI = jax.lax.Precision.HIGHEST


def _sc_gather(tok_emb, idx):
    n = idx.shape[1]
    d = tok_emb.shape[1]
    n_units = 2 * 16
    per_unit = n // n_units
    chunk = 64
    mesh = plsc.VectorSubcoreMesh(core_axis_name="c", subcore_axis_name="s")

    @partial(
        pl.kernel,
        out_type=jax.ShapeDtypeStruct((n, d), tok_emb.dtype),
        mesh=mesh,
        scratch_types=[
            pltpu.VMEM((1, per_unit), jnp.int32),
            pltpu.VMEM((chunk, d), jnp.float32),
        ],
    )
    def gather_kernel(emb_hbm, idx_hbm, out_hbm, idx_vmem, buf):
        c = jax.lax.axis_index("c")
        s = jax.lax.axis_index("s")
        base = (c * 16 + s) * per_unit
        pltpu.sync_copy(idx_hbm.at[:, pl.ds(base, per_unit)], idx_vmem)

        @pl.loop(0, per_unit // chunk)
        def _(j):
            pltpu.sync_copy(emb_hbm.at[idx_vmem.at[0, pl.ds(j * chunk, chunk)]],
                            buf)
            pltpu.sync_copy(buf, out_hbm.at[pl.ds(base + j * chunk, chunk), :])

    return gather_kernel(tok_emb, idx)


def _proj_body(img_ref, wimg_ref, aud_ref, waud_ref, bias_ref, out_ref):
    i = pl.program_id(0)

    @pl.when(i == 0)
    def _init():
        out_ref[...] = jnp.broadcast_to(bias_ref[...], (_B, _D))

    @pl.when(i < _NB_IMG)
    def _img():
        w = wimg_ref[...]
        it = jnp.transpose(img_ref[...])
        for b in range(_B):
            out_ref[b:b + 1, :] += jnp.sum(
                w * it[:, b:b + 1], axis=0, keepdims=True)

    @pl.when(i >= _NB_IMG)
    def _aud():
        w = waud_ref[...]
        at = jnp.transpose(aud_ref[...])
        for b in range(_B):
            out_ref[b:b + 1, :] += jnp.sum(
                w * at[:, b:b + 1], axis=0, keepdims=True)


def _proj(img, W_img, aud, W_aud, bias_row):
    grid = (_NB_IMG + _NB_AUD,)
    return pl.pallas_call(
        _proj_body,
        grid=grid,
        in_specs=[
            pl.BlockSpec((_B, _KB_IMG), lambda i: (0, jnp.minimum(i, _NB_IMG - 1))),
            pl.BlockSpec((_KB_IMG, _D), lambda i: (jnp.minimum(i, _NB_IMG - 1), 0)),
            pl.BlockSpec((_B, _KB_AUD), lambda i: (0, jnp.maximum(i - _NB_IMG, 0))),
            pl.BlockSpec((_KB_AUD, _D), lambda i: (jnp.maximum(i - _NB_IMG, 0), 0)),
            pl.BlockSpec((1, _D), lambda i: (0, 0)),
        ],
        out_specs=pl.BlockSpec((_B, _D), lambda i: (0, 0)),
        out_shape=jax.ShapeDtypeStruct((_B, _D), jnp.float32),
    )(img, W_img, aud, W_aud, bias_row)


def _moe_body(t_ref, m_ref, c_ref, wr_ref, wexp_ref, bexp_ref, out_ref):
    i = pl.program_id(0)
    m = m_ref[:, 0:1]
    cval = c_ref[...]
    b_idx = (i * _SB) // _S
    crow = jnp.where(b_idx == 0, cval[0:1, :], cval[1:2, :])
    x = t_ref[...] * m + crow

    logits = jax.lax.dot_general(
        x, wr_ref[...], (((1,), (0,)), ((), ())),
        precision=_HI, preferred_element_type=jnp.float32)
    col = jax.lax.broadcasted_iota(jnp.int32, (_SB, _EPAD), 1)
    valid = col < _E
    neg = jnp.where(valid, logits, -jnp.inf)
    mx = jnp.max(neg, axis=1, keepdims=True)
    ex = jnp.where(valid, jnp.exp(neg - mx), 0.0)
    gates = ex / jnp.sum(ex, axis=1, keepdims=True)

    acc = jax.lax.dot_general(
        gates, bexp_ref[...], (((1,), (0,)), ((), ())),
        precision=_HI, preferred_element_type=jnp.float32)
    x_bf = x.astype(jnp.bfloat16)
    for e in range(_E):
        y = jax.lax.dot_general(
            x_bf, wexp_ref[e], (((1,), (0,)), ((), ())),
            preferred_element_type=jnp.float32)
        acc = acc + gates[:, e:e + 1] * y
    out_ref[...] = acc * m


def _moe(t, maskb, c, wr_pad, wexp, bexp_pad):
    grid = (_N // _SB,)
    return pl.pallas_call(
        _moe_body,
        grid=grid,
        in_specs=[
            pl.BlockSpec((_SB, _D), lambda i: (i, 0)),
            pl.BlockSpec((_SB, _EPAD), lambda i: (i, 0)),
            pl.BlockSpec((_B, _D), lambda i: (0, 0)),
            pl.BlockSpec((_D, _EPAD), lambda i: (0, 0)),
            pl.BlockSpec((_E, _D, _D), lambda i: (0, 0, 0)),
            pl.BlockSpec((_EPAD, _D), lambda i: (0, 0)),
        ],
        out_specs=pl.BlockSpec((_SB, _D), lambda i: (i, 0)),
        out_shape=jax.ShapeDtypeStruct((_N, _D), jnp.float32),
    )(t, maskb, c, wr_pad, wexp, bexp_pad)


def kernel(text_input, attention_mask, image_input, audio_input, tok_emb,
           W_img, b_img, W_aud, b_aud, W_router, W_experts, b_experts):
    idx = text_input.reshape(1, _N).astype(jnp.int32)
    t = _sc_gather(tok_emb, idx)

    img = image_input.reshape(_B, -1)
    bias_row = (b_img + b_aud).reshape(1, _D)
    c = _proj(img, W_img, audio_input, W_aud, bias_row)

    maskb = jnp.broadcast_to(
        attention_mask.astype(jnp.float32).reshape(_N, 1), (_N, _EPAD))
    wr_pad = jnp.zeros((_D, _EPAD), jnp.float32).at[:, :_E].set(W_router)
    bexp_pad = jnp.zeros((_EPAD, _D), jnp.float32).at[:_E, :].set(b_experts)

    out = _moe(t, maskb, c, wr_pad, W_experts.astype(jnp.bfloat16), bexp_pad)
    return out.reshape(_B, _S, _D)


# speedup vs baseline: 2.3115x; 1.0933x over previous
from functools import partial

import jax
import jax.numpy as jnp
from jax.experimental import pallas as pl
from jax.experimental.pallas import tpu as pltpu
from jax.experimental.pallas import tpu_sc as plsc

_B, _S, _D, _E, _V = 2, 2048, 768, 8, 100000
_N = _B * _S
_KB_IMG = 3072
_NB_IMG = 49
_KB_AUD = 3200
_NB_AUD = 5
_SB = 1024
_EPAD = 128

_HI = jax.lax.Precision.HIGHEST


def _sc_gather(tok_emb, idx):
    n = idx.shape[1]
    d = tok_emb.shape[1]
    n_units = 2 * 16
    per_unit = n // n_units
    chunk = 64
    mesh = plsc.VectorSubcoreMesh(core_axis_name="c", subcore_axis_name="s")

    @partial(
        pl.kernel,
        out_type=jax.ShapeDtypeStruct((n, d), tok_emb.dtype),
        mesh=mesh,
        scratch_types=[
            pltpu.VMEM((1, per_unit), jnp.int32),
            pltpu.VMEM((chunk, d), jnp.float32),
        ],
    )
    def gather_kernel(emb_hbm, idx_hbm, out_hbm, idx_vmem, buf):
        c = jax.lax.axis_index("c")
        s = jax.lax.axis_index("s")
        base = (c * 16 + s) * per_unit
        pltpu.sync_copy(idx_hbm.at[:, pl.ds(base, per_unit)], idx_vmem)

        @pl.loop(0, per_unit // chunk)
        def _(j):
            pltpu.sync_copy(emb_hbm.at[idx_vmem.at[0, pl.ds(j * chunk, chunk)]],
                            buf)
            pltpu.sync_copy(buf, out_hbm.at[pl.ds(base + j * chunk, chunk), :])

    return gather_kernel(tok_emb, idx)


def _proj_body(img_ref, wimg_ref, aud_ref, waud_ref, bias_ref, out_ref):
    i = pl.program_id(0)

    @pl.when(i == 0)
    def _init():
        out_ref[...] = jnp.broadcast_to(bias_ref[...], (_B, _D))

    @pl.when(i < _NB_IMG)
    def _img():
        w = wimg_ref[...]
        it = jnp.transpose(img_ref[...])
        for b in range(_B):
            out_ref[b:b + 1, :] += jnp.sum(
                w * it[:, b:b + 1], axis=0, keepdims=True)

    @pl.when(i >= _NB_IMG)
    def _aud():
        w = waud_ref[...]
        at = jnp.transpose(aud_ref[...])
        for b in range(_B):
            out_ref[b:b + 1, :] += jnp.sum(
                w * at[:, b:b + 1], axis=0, keepdims=True)


def _proj(img, W_img, aud, W_aud, bias_row):
    grid = (_NB_IMG + _NB_AUD,)
    return pl.pallas_call(
        _proj_body,
        grid=grid,
        in_specs=[
            pl.BlockSpec((_B, _KB_IMG), lambda i: (0, jnp.minimum(i, _NB_IMG - 1))),
            pl.BlockSpec((_KB_IMG, _D), lambda i: (jnp.minimum(i, _NB_IMG - 1), 0)),
            pl.BlockSpec((_B, _KB_AUD), lambda i: (0, jnp.maximum(i - _NB_IMG, 0))),
            pl.BlockSpec((_KB_AUD, _D), lambda i: (jnp.maximum(i - _NB_IMG, 0), 0)),
            pl.BlockSpec((1, _D), lambda i: (0, 0)),
        ],
        out_specs=pl.BlockSpec((_B, _D), lambda i: (0, 0)),
        out_shape=jax.ShapeDtypeStruct((_B, _D), jnp.float32),
    )(img, W_img, aud, W_aud, bias_row)


def _moe_body(t_ref, m_ref, c_ref, wr_ref, wrbf_ref, wexp_ref, bexp_ref,
              out_ref, cb_scr, clog_scr):
    i = pl.program_id(0)

    @pl.when(i == 0)
    def _init():
        cval = c_ref[...]
        clog_scr[0:_B, :] = jax.lax.dot_general(
            cval, wr_ref[...], (((1,), (0,)), ((), ())),
            precision=_HI, preferred_element_type=jnp.float32)
        bex = bexp_ref[...]
        cbf = cval.astype(jnp.bfloat16)
        for b in range(_B):
            cb_scr[b] = bex.astype(jnp.bfloat16)
        for e in range(_E):
            r = jax.lax.dot_general(
                cbf, wexp_ref[e], (((1,), (0,)), ((), ())),
                preferred_element_type=jnp.float32)
            for b in range(_B):
                cb_scr[b, e:e + 1, :] = (
                    r[b:b + 1, :] + bex[e:e + 1, :]).astype(jnp.bfloat16)

    m = m_ref[:, 0:1]
    b_idx = i // (_S // _SB)
    tm = t_ref[...] * m
    tm_bf = tm.astype(jnp.bfloat16)

    clog_row = jnp.where(b_idx == 0, clog_scr[0:1, :], clog_scr[1:2, :])
    logits = jax.lax.dot_general(
        tm_bf, wrbf_ref[...], (((1,), (0,)), ((), ())),
        preferred_element_type=jnp.float32) + clog_row
    col = jax.lax.broadcasted_iota(jnp.int32, (_SB, _EPAD), 1)
    valid = col < _E
    neg = jnp.where(valid, logits, -jnp.inf)
    mx = jnp.max(neg, axis=1, keepdims=True)
    ex = jnp.where(valid, jnp.exp(neg - mx), 0.0)
    gates = ex / jnp.sum(ex, axis=1, keepdims=True)

    cb_b = jnp.where(b_idx == 0, cb_scr[0], cb_scr[1])
    acc = jax.lax.dot_general(
        gates.astype(jnp.bfloat16), cb_b, (((1,), (0,)), ((), ())),
        preferred_element_type=jnp.float32)
    for e in range(_E):
        y = jax.lax.dot_general(
            tm_bf, wexp_ref[e], (((1,), (0,)), ((), ())),
            preferred_element_type=jnp.float32)
        acc = acc + gates[:, e:e + 1] * y
    out_ref[...] = acc * m


def _moe(t, maskb, c, wr_pad, wrbf, wexp_bf, bexp_pad):
    grid = (_N // _SB,)
    return pl.pallas_call(
        _moe_body,
        grid=grid,
        in_specs=[
            pl.BlockSpec((_SB, _D), lambda i: (i, 0)),
            pl.BlockSpec((_SB, _EPAD), lambda i: (i, 0)),
            pl.BlockSpec((_B, _D), lambda i: (0, 0)),
            pl.BlockSpec((_D, _EPAD), lambda i: (0, 0)),
            pl.BlockSpec((_D, _EPAD), lambda i: (0, 0)),
            pl.BlockSpec((_E, _D, _D), lambda i: (0, 0, 0)),
            pl.BlockSpec((_EPAD, _D), lambda i: (0, 0)),
        ],
        out_specs=pl.BlockSpec((_SB, _D), lambda i: (i, 0)),
        out_shape=jax.ShapeDtypeStruct((_N, _D), jnp.float32),
        scratch_shapes=[
            pltpu.VMEM((_B, _EPAD, _D), jnp.bfloat16),
            pltpu.VMEM((8, _EPAD), jnp.float32),
        ],
    )(t, maskb, c, wr_pad, wrbf, wexp_bf, bexp_pad)


def kernel(text_input, attention_mask, image_input, audio_input, tok_emb,
           W_img, b_img, W_aud, b_aud, W_router, W_experts, b_experts):
    idx = text_input.reshape(1, _N).astype(jnp.int32)
    t = _sc_gather(tok_emb, idx)

    img = image_input.reshape(_B, -1)
    bias_row = (b_img + b_aud).reshape(1, _D)
    c = _proj(img, W_img, audio_input, W_aud, bias_row)

    maskb = jnp.broadcast_to(
        attention_mask.astype(jnp.float32).reshape(_N, 1), (_N, _EPAD))
    wr_pad = jnp.zeros((_D, _EPAD), jnp.float32).at[:, :_E].set(W_router)
    bexp_pad = jnp.zeros((_EPAD, _D), jnp.float32).at[:_E, :].set(b_experts)

    out = _moe(t, maskb, c, wr_pad, wr_pad.astype(jnp.bfloat16),
               W_experts.astype(jnp.bfloat16), bexp_pad)
    return out.reshape(_B, _S, _D)
